# baseline (speedup 1.0000x reference)
import functools
import math

import jax
import jax.numpy as jnp
from jax import lax
from jax.experimental import pallas as pl
from jax.experimental.pallas import tpu as pltpu
from jax.experimental.pallas import tpu_sc as plsc

N_HEADS = 16
QLEN = 2048
KLEN = 2048
TBL = 4224


def _table_body(bias_ref, out_ref):
    i = lax.broadcasted_iota(jnp.int32, (1, TBL), 1)
    delta = i - 2047
    rel_gt = (delta > 0).astype(jnp.int32) * 128
    ad = jnp.abs(delta)
    is_small = ad < 64
    ril = 64 + (jnp.log(ad.astype(jnp.float32) / 64)
                / math.log(2048 / 64) * (128 - 64)).astype(jnp.int32)
    ril = jnp.minimum(ril, jnp.full_like(ril, 127))
    inner = rel_gt + jnp.where(is_small, ad, ril)
    bucket = jnp.where(i < 4095, inner, jnp.minimum(i - 4095 + 256, 287))
    onehot = (bucket == lax.broadcasted_iota(jnp.int32, (288, TBL), 0))
    out_ref[...] = lax.dot_general(
        bias_ref[...], onehot.astype(jnp.float32),
        (((0,), (0,)), ((), ())), preferred_element_type=jnp.float32,
        precision=lax.Precision.HIGHEST)


def _build_table(bias):
    return pl.pallas_call(
        _table_body,
        out_shape=jax.ShapeDtypeStruct((N_HEADS, TBL), jnp.float32),
    )(bias)


KQ = 512
N_UNITS = 64


def _sc_body(tt_hbm, qpos_hbm, kpos_hbm, rel_hbm, out_hbm,
             t_v, kpos_v, qpos_v, rel_v, out_v,
             sem_r0, sem_r1, sem_o0, sem_o1):
    nc = 2
    wid = lax.axis_index("s") * nc + lax.axis_index("c")
    hh = wid % 2
    rb = wid // 2
    h0 = hh * 8
    q0 = rb * 128
    sem_r = (sem_r0, sem_r1)
    sem_o = (sem_o0, sem_o1)

    def unit_sm(u):
        return u // 4, u % 4

    def rel_copy(u, p):
        s, m = unit_sm(u)
        return pltpu.make_async_copy(
            rel_hbm.at[0, pl.ds(q0 + 8 * s, 8), pl.ds(KQ * m, KQ)],
            rel_v.at[p], sem_r[p])

    def out_copy(u, p):
        s, m = unit_sm(u)
        return pltpu.make_async_copy(
            out_v.at[p],
            out_hbm.at[0, pl.ds(h0, 8), pl.ds(q0 + 8 * s, 8),
                       pl.ds(KQ * m, KQ)],
            sem_o[p])

    pltpu.sync_copy(tt_hbm.at[pl.ds(h0, 8)], t_v)
    rel_copy(0, 0).start()
    rel_copy(1, 1).start()
    pltpu.sync_copy(kpos_hbm, kpos_v)
    pltpu.sync_copy(qpos_hbm, qpos_v)

    def unit_pair(i, carry):
        for p in range(2):
            u = i * 2 + p
            s, m = unit_sm(u)

            @pl.when(u >= 2)
            def _wait_out():
                out_copy(u - 2, p).wait()

            rel_copy(u, p).wait()

            def row(qi, carry2):
                qsplat = plsc.load_gather(
                    qpos_v,
                    [jnp.full((16,), 0, jnp.int32) + (q0 + 8 * s + qi)])
                qoff = 2047 - qsplat

                @plsc.parallel_loop(0, KQ // 16, unroll=16)
                def chunk(c):
                    kp = kpos_v[pl.ds(KQ * m + c * 16, 16)]
                    rel = rel_v[p, qi, pl.ds(c * 16, 16)]
                    idx = jnp.where(rel == 0, kp + qoff, rel + 4094)
                    for h in range(8):
                        val = plsc.load_gather(
                            t_v, [jnp.full((16,), h, jnp.int32), idx])
                        out_v[p, h, qi, pl.ds(c * 16, 16)] = val

                return carry2

            lax.fori_loop(0, 8, row, 0)
            out_copy(u, p).start()

            @pl.when(u + 2 < N_UNITS)
            def _prefetch_rel():
                rel_copy(u + 2, p).start()
        return carry

    lax.fori_loop(0, N_UNITS // 2, unit_pair, 0)
    out_copy(N_UNITS - 2, 0).wait()
    out_copy(N_UNITS - 1, 1).wait()


@functools.partial(jax.jit, static_argnums=())
def _sc_gather(tt, qpos, kpos, rel):
    fn = pl.kernel(
        _sc_body,
        mesh=plsc.VectorSubcoreMesh(core_axis_name="c", subcore_axis_name="s"),
        out_type=jax.ShapeDtypeStruct((1, N_HEADS, QLEN, KLEN), jnp.float32),
        compiler_params=pltpu.CompilerParams(needs_layout_passes=False),
        scratch_types=[
            pltpu.VMEM((8, TBL), jnp.float32),
            pltpu.VMEM((KLEN,), jnp.int32),
            pltpu.VMEM((QLEN,), jnp.int32),
            pltpu.VMEM((2, 8, KQ), jnp.int32),
            pltpu.VMEM((2, 8, 8, KQ), jnp.float32),
            pltpu.SemaphoreType.DMA,
            pltpu.SemaphoreType.DMA,
            pltpu.SemaphoreType.DMA,
            pltpu.SemaphoreType.DMA,
        ],
    )
    return fn(tt, qpos, kpos, rel)


def kernel(query_pos, key_pos, rel_buckets, relative_attention_bias):
    qp = query_pos.reshape(QLEN)
    kp = key_pos.reshape(KLEN)
    tt = _build_table(relative_attention_bias)
    return _sc_gather(tt, qp, kp, rel_buckets)


# speedup vs baseline: 1.0905x; 1.0905x over previous
import functools
import math

import jax
import jax.numpy as jnp
from jax import lax
from jax.experimental import pallas as pl
from jax.experimental.pallas import tpu as pltpu
from jax.experimental.pallas import tpu_sc as plsc

N_HEADS = 16
QLEN = 2048
KLEN = 2048
TBL = 4224


def _table_body(bias_ref, out_ref):
    i = lax.broadcasted_iota(jnp.int32, (1, TBL), 1)
    delta = i - 2047
    rel_gt = (delta > 0).astype(jnp.int32) * 128
    ad = jnp.abs(delta)
    is_small = ad < 64
    ril = 64 + (jnp.log(ad.astype(jnp.float32) / 64)
                / math.log(2048 / 64) * (128 - 64)).astype(jnp.int32)
    ril = jnp.minimum(ril, jnp.full_like(ril, 127))
    inner = rel_gt + jnp.where(is_small, ad, ril)
    bucket = jnp.where(i < 4095, inner, jnp.minimum(i - 4095 + 256, 287))
    onehot = (bucket == lax.broadcasted_iota(jnp.int32, (288, TBL), 0))
    out_ref[...] = lax.dot_general(
        bias_ref[...], onehot.astype(jnp.float32),
        (((0,), (0,)), ((), ())), preferred_element_type=jnp.float32,
        precision=lax.Precision.HIGHEST)


def _build_table(bias):
    return pl.pallas_call(
        _table_body,
        out_shape=jax.ShapeDtypeStruct((N_HEADS, TBL), jnp.float32),
    )(bias)


KQ = 512
N_UNITS = 64


def _sc_body(tt_hbm, qpos_hbm, kpos_hbm, rel_hbm, out_hbm,
             t_v, kpos_v, qpos_v, rel_v, out_v,
             sem_r0, sem_r1, sem_o0, sem_o1):
    nc = 2
    wid = lax.axis_index("s") * nc + lax.axis_index("c")
    hh = wid % 2
    rb = wid // 2
    h0 = hh * 8
    q0 = rb * 128
    sem_r = (sem_r0, sem_r1)
    sem_o = (sem_o0, sem_o1)

    def unit_sm(u):
        return u // 4, u % 4

    def rel_copy(u, p):
        s, m = unit_sm(u)
        return pltpu.make_async_copy(
            rel_hbm.at[0, pl.ds(q0 + 8 * s, 8), pl.ds(KQ * m, KQ)],
            rel_v.at[p], sem_r[p])

    def out_copy(u, p):
        s, m = unit_sm(u)
        return pltpu.make_async_copy(
            out_v.at[p],
            out_hbm.at[0, pl.ds(h0, 8), pl.ds(q0 + 8 * s, 8),
                       pl.ds(KQ * m, KQ)],
            sem_o[p])

    tt_dma = pltpu.make_async_copy(tt_hbm.at[pl.ds(h0, 8)], t_v, sem_o0)
    tt_dma.start()
    rel_copy(0, 0).start()
    rel_copy(1, 1).start()
    pltpu.sync_copy(kpos_hbm, kpos_v)
    pltpu.sync_copy(qpos_hbm, qpos_v)
    tt_dma.wait()

    def unit_pair(i, carry):
        for p in range(2):
            u = i * 2 + p
            s, m = unit_sm(u)

            @pl.when(u >= 2)
            def _wait_out():
                out_copy(u - 2, p).wait()

            rel_copy(u, p).wait()

            def row(qi, carry2):
                qsplat = plsc.load_gather(
                    qpos_v,
                    [jnp.full((16,), 0, jnp.int32) + (q0 + 8 * s + qi)])
                qoff = 2047 - qsplat

                @plsc.parallel_loop(0, KQ // 16, unroll=8)
                def chunk(c):
                    kp = kpos_v[pl.ds(KQ * m + c * 16, 16)]
                    rel = rel_v[p, qi, pl.ds(c * 16, 16)]
                    idx = jnp.where(rel == 0, kp + qoff, rel + 4094)
                    for h in range(8):
                        val = plsc.load_gather(
                            t_v, [jnp.full((16,), h, jnp.int32), idx])
                        out_v[p, h, qi, pl.ds(c * 16, 16)] = val

                return carry2

            lax.fori_loop(0, 8, row, 0)
            out_copy(u, p).start()

            @pl.when(u + 2 < N_UNITS)
            def _prefetch_rel():
                rel_copy(u + 2, p).start()
        return carry

    lax.fori_loop(0, N_UNITS // 2, unit_pair, 0)
    out_copy(N_UNITS - 2, 0).wait()
    out_copy(N_UNITS - 1, 1).wait()


@functools.partial(jax.jit, static_argnums=())
def _sc_gather(tt, qpos, kpos, rel):
    fn = pl.kernel(
        _sc_body,
        mesh=plsc.VectorSubcoreMesh(core_axis_name="c", subcore_axis_name="s"),
        out_type=jax.ShapeDtypeStruct((1, N_HEADS, QLEN, KLEN), jnp.float32),
        compiler_params=pltpu.CompilerParams(needs_layout_passes=False),
        scratch_types=[
            pltpu.VMEM((8, TBL), jnp.float32),
            pltpu.VMEM((KLEN,), jnp.int32),
            pltpu.VMEM((QLEN,), jnp.int32),
            pltpu.VMEM((2, 8, KQ), jnp.int32),
            pltpu.VMEM((2, 8, 8, KQ), jnp.float32),
            pltpu.SemaphoreType.DMA,
            pltpu.SemaphoreType.DMA,
            pltpu.SemaphoreType.DMA,
            pltpu.SemaphoreType.DMA,
        ],
    )
    return fn(tt, qpos, kpos, rel)


def kernel(query_pos, key_pos, rel_buckets, relative_attention_bias):
    qp = query_pos.reshape(QLEN)
    kp = key_pos.reshape(KLEN)
    tt = _build_table(relative_attention_bias)
    return _sc_gather(tt, qp, kp, rel_buckets)
